# baseline (speedup 1.0000x reference)
import jax
import jax.numpy as jnp
from jax.experimental import pallas as pl
from jax.experimental.pallas import tpu as pltpu

B = 2
Q = 150
C = 150
K = 100
KP = 128
HW = 384 * 384
BN = 8192
NBLK = HW // BN
BIGI = 2147483647


def _topk_body(logits_ref, probs_t_ref, vals_ref, labels_ref, oh_ref, oht_ref):
    lg = logits_ref[...]
    m = jnp.max(lg, axis=-1, keepdims=True)
    e = jnp.exp(lg - m)
    p = e / jnp.sum(e, axis=-1, keepdims=True)
    scores = p[:, :, :C]
    probs_t_ref[...] = jnp.swapaxes(scores, 1, 2)

    fi = (jax.lax.broadcasted_iota(jnp.int32, (1, Q, C), 1) * C
          + jax.lax.broadcasted_iota(jnp.int32, (1, Q, C), 2))
    li = jax.lax.broadcasted_iota(jnp.int32, (1, 1, KP), 2)

    def step(i, carry):
        x, acc_v, acc_i = carry
        mv = jnp.max(jnp.max(x, axis=2, keepdims=True), axis=1, keepdims=True)
        cand = jnp.where(x == mv, fi, BIGI)
        fidx = jnp.min(jnp.min(cand, axis=2, keepdims=True), axis=1,
                       keepdims=True)
        x = jnp.where(fi == fidx, -1.0, x)
        acc_v = jnp.where(li == i, mv, acc_v)
        acc_i = jnp.where(li == i, fidx, acc_i)
        return x, acc_v, acc_i

    acc_v0 = jnp.zeros((B, 1, KP), jnp.float32)
    acc_i0 = jnp.zeros((B, 1, KP), jnp.int32)
    _, acc_v, acc_i = jax.lax.fori_loop(0, K, step, (scores, acc_v0, acc_i0))

    vals_ref[...] = acc_v
    labels_ref[...] = acc_i % C
    qv = acc_i // C
    kmask = li < K
    qi = jax.lax.broadcasted_iota(jnp.int32, (1, Q, 1), 1)
    oht = jnp.where((qv == qi) & kmask, 1.0, 0.0)
    oht_ref[...] = oht
    oh_ref[...] = jnp.swapaxes(oht, 1, 2)


def _mega_body(masks_ref, pt_ref, oh_ref, oht_ref, vals_ref,
               sem_ref, inst_ref, num_ref, den_ref, fin_ref):
    n = pl.program_id(1)
    x = masks_ref[0]
    sig = jax.nn.sigmoid(x)
    ind = jnp.where(x > 0, 1.0, 0.0)
    sem_ref[0] = jnp.dot(pt_ref[0], sig,
                         preferred_element_type=jnp.float32)
    io = jnp.dot(oh_ref[0], ind, preferred_element_type=jnp.float32)
    inst_ref[0] = io[:K]
    pn = jnp.sum(jnp.where(x > 0, sig, 0.0), axis=1, keepdims=True)
    pd = jnp.sum(ind, axis=1, keepdims=True)

    @pl.when(n == 0)
    def _():
        num_ref[0] = pn
        den_ref[0] = pd

    @pl.when(n > 0)
    def _():
        num_ref[0] += pn
        den_ref[0] += pd

    @pl.when(n == NBLK - 1)
    def _():
        ratio = num_ref[0] / (den_ref[0] + 1e-6)
        rs = jnp.dot(ratio.reshape(1, Q), oht_ref[0],
                     preferred_element_type=jnp.float32)
        fin_ref[0] = vals_ref[0] * rs


def kernel(pred_logits, pred_masks):
    probs_t, vals, labels, oh, oht = pl.pallas_call(
        _topk_body,
        out_shape=(
            jax.ShapeDtypeStruct((B, C, Q), jnp.float32),
            jax.ShapeDtypeStruct((B, 1, KP), jnp.float32),
            jax.ShapeDtypeStruct((B, 1, KP), jnp.int32),
            jax.ShapeDtypeStruct((B, KP, Q), jnp.float32),
            jax.ShapeDtypeStruct((B, Q, KP), jnp.float32),
        ),
    )(pred_logits)

    masks_flat = pred_masks.reshape(B, Q, HW)
    sem_flat, inst_flat, num, den, fin = pl.pallas_call(
        _mega_body,
        grid=(B, NBLK),
        in_specs=[
            pl.BlockSpec((1, Q, BN), lambda b, n: (b, 0, n)),
            pl.BlockSpec((1, C, Q), lambda b, n: (b, 0, 0)),
            pl.BlockSpec((1, KP, Q), lambda b, n: (b, 0, 0)),
            pl.BlockSpec((1, Q, KP), lambda b, n: (b, 0, 0)),
            pl.BlockSpec((1, 1, KP), lambda b, n: (b, 0, 0)),
        ],
        out_specs=(
            pl.BlockSpec((1, C, BN), lambda b, n: (b, 0, n)),
            pl.BlockSpec((1, K, BN), lambda b, n: (b, 0, n)),
            pl.BlockSpec((1, Q, 1), lambda b, n: (b, 0, 0)),
            pl.BlockSpec((1, Q, 1), lambda b, n: (b, 0, 0)),
            pl.BlockSpec((1, 1, KP), lambda b, n: (b, 0, 0)),
        ),
        out_shape=(
            jax.ShapeDtypeStruct((B, C, HW), jnp.float32),
            jax.ShapeDtypeStruct((B, K, HW), jnp.float32),
            jax.ShapeDtypeStruct((B, Q, 1), jnp.float32),
            jax.ShapeDtypeStruct((B, Q, 1), jnp.float32),
            jax.ShapeDtypeStruct((B, 1, KP), jnp.float32),
        ),
        compiler_params=pltpu.CompilerParams(
            dimension_semantics=("parallel", "arbitrary")),
    )(masks_flat, probs_t, oh, oht, vals)

    semseg = sem_flat.reshape(B, C, 384, 384)
    inst = inst_flat.reshape(B, K, 384, 384)
    final_scores = fin[:, 0, :K]
    labels_out = labels[:, 0, :K]
    return semseg, inst, final_scores, labels_out


# speedup vs baseline: 20.7612x; 20.7612x over previous
import dataclasses

import jax
import jax.numpy as jnp
from jax.experimental import pallas as pl
from jax.experimental.pallas import tpu as pltpu
from jax.experimental.pallas import tpu_sc as plsc

B = 2
Q = 150
C = 150
K = 100
KP = 128
H = 384
W = 384
BH = 32
NBLK = H // BH
BIGI = 2147483647
QP = 160


def _topk_body(logits_ref, probs_t_ref, vals_ref, labels_ref, q_ref, oh_ref):
    lg = logits_ref[...]
    m = jnp.max(lg, axis=-1, keepdims=True)
    e = jnp.exp(lg - m)
    p = e / jnp.sum(e, axis=-1, keepdims=True)
    scores = p[:, :, :C]
    probs_t_ref[...] = jnp.swapaxes(scores, 1, 2)

    fi = (jax.lax.broadcasted_iota(jnp.int32, (1, Q, C), 1) * C
          + jax.lax.broadcasted_iota(jnp.int32, (1, Q, C), 2))
    li = jax.lax.broadcasted_iota(jnp.int32, (1, 1, KP), 2)

    def step(i, carry):
        x, acc_v, acc_i = carry
        mv = jnp.max(jnp.max(x, axis=2, keepdims=True), axis=1, keepdims=True)
        cand = jnp.where(x == mv, fi, BIGI)
        fidx = jnp.min(jnp.min(cand, axis=2, keepdims=True), axis=1,
                       keepdims=True)
        x = jnp.where(fi == fidx, -1.0, x)
        acc_v = jnp.where(li == i, mv, acc_v)
        acc_i = jnp.where(li == i, fidx, acc_i)
        return x, acc_v, acc_i

    acc_v0 = jnp.zeros((B, 1, KP), jnp.float32)
    acc_i0 = jnp.zeros((B, 1, KP), jnp.int32)
    _, acc_v, acc_i = jax.lax.fori_loop(0, K, step, (scores, acc_v0, acc_i0))

    vals_ref[...] = acc_v
    labels_ref[...] = acc_i % C
    qv = acc_i // C
    q_ref[...] = qv
    kmask = li < K
    qi = jax.lax.broadcasted_iota(jnp.int32, (1, Q, 1), 1)
    oht = jnp.where((qv == qi) & kmask, 1.0, 0.0)
    oh_ref[...] = jnp.swapaxes(oht, 1, 2)


def _mega_body(masks_ref, pt_ref, oh_ref,
               sem_ref, inst_ref, num_ref, den_ref):
    n = pl.program_id(1)
    xb = masks_ref[0].astype(jnp.bfloat16).reshape(Q, BH * W)
    sig = jax.nn.sigmoid(xb)
    one = jnp.ones((), jnp.bfloat16)
    ind = jnp.where(xb > 0, one, jnp.zeros((), jnp.bfloat16))
    sem2 = jnp.dot(pt_ref[0].astype(jnp.bfloat16), sig,
                   preferred_element_type=jnp.float32)
    sem_ref[0] = sem2.reshape(C, BH, W)
    io = jnp.dot(oh_ref[0].astype(jnp.bfloat16), ind,
                 preferred_element_type=jnp.float32)
    inst_ref[0] = io[:K].reshape(K, BH, W)
    masked = jnp.where(xb > 0, sig, jnp.zeros((), jnp.bfloat16))
    pn = jnp.sum(masked, axis=1, keepdims=True, dtype=jnp.float32)
    pd = jnp.sum(ind, axis=1, keepdims=True, dtype=jnp.float32)

    @pl.when(n == 0)
    def _():
        num_ref[0] = pn
        den_ref[0] = pd

    @pl.when(n > 0)
    def _():
        num_ref[0] += pn
        den_ref[0] += pd


def _sc_final(vals2, q2, numpad, denpad):
    vec_mesh = plsc.VectorSubcoreMesh(core_axis_name="c",
                                      subcore_axis_name="s")
    cp = pltpu.CompilerParams()
    if "needs_layout_passes" in pltpu.CompilerParams.__dataclass_fields__:
        cp = dataclasses.replace(cp, needs_layout_passes=False)

    @pl.kernel(
        out_type=jax.ShapeDtypeStruct((B, KP), jnp.float32),
        mesh=vec_mesh,
        compiler_params=cp,
        scratch_types=[
            pltpu.VMEM((16,), jnp.float32),
            pltpu.VMEM((16,), jnp.int32),
            pltpu.VMEM((QP,), jnp.float32),
            pltpu.VMEM((QP,), jnp.float32),
            pltpu.VMEM((16,), jnp.float32),
        ],
    )
    def sc_kernel(vals_hbm, q_hbm, num_hbm, den_hbm, o_hbm,
                  sv, sq, sn, sd, so):
        b = jax.lax.axis_index("c")
        s = jax.lax.axis_index("s")

        @pl.when(s < 8)
        def _():
            off = s * 16
            pltpu.sync_copy(vals_hbm.at[b, pl.ds(off, 16)], sv)
            pltpu.sync_copy(q_hbm.at[b, pl.ds(off, 16)], sq)
            pltpu.sync_copy(num_hbm.at[b], sn)
            pltpu.sync_copy(den_hbm.at[b], sd)
            idx = sq[...]
            gn = plsc.load_gather(sn, (idx,))
            gd = plsc.load_gather(sd, (idx,))
            so[...] = sv[...] * gn / (gd + 1e-6)
            pltpu.sync_copy(so, o_hbm.at[b, pl.ds(off, 16)])

    return sc_kernel(vals2, q2, numpad, denpad)


def kernel(pred_logits, pred_masks):
    probs_t, vals, labels, qidx, oh = pl.pallas_call(
        _topk_body,
        out_shape=(
            jax.ShapeDtypeStruct((B, C, Q), jnp.float32),
            jax.ShapeDtypeStruct((B, 1, KP), jnp.float32),
            jax.ShapeDtypeStruct((B, 1, KP), jnp.int32),
            jax.ShapeDtypeStruct((B, 1, KP), jnp.int32),
            jax.ShapeDtypeStruct((B, KP, Q), jnp.float32),
        ),
    )(pred_logits)

    sem, inst, num, den = pl.pallas_call(
        _mega_body,
        grid=(B, NBLK),
        in_specs=[
            pl.BlockSpec((1, Q, BH, W), lambda b, n: (b, 0, n, 0)),
            pl.BlockSpec((1, C, Q), lambda b, n: (b, 0, 0)),
            pl.BlockSpec((1, KP, Q), lambda b, n: (b, 0, 0)),
        ],
        out_specs=(
            pl.BlockSpec((1, C, BH, W), lambda b, n: (b, 0, n, 0)),
            pl.BlockSpec((1, K, BH, W), lambda b, n: (b, 0, n, 0)),
            pl.BlockSpec((1, Q, 1), lambda b, n: (b, 0, 0)),
            pl.BlockSpec((1, Q, 1), lambda b, n: (b, 0, 0)),
        ),
        out_shape=(
            jax.ShapeDtypeStruct((B, C, H, W), jnp.float32),
            jax.ShapeDtypeStruct((B, K, H, W), jnp.float32),
            jax.ShapeDtypeStruct((B, Q, 1), jnp.float32),
            jax.ShapeDtypeStruct((B, Q, 1), jnp.float32),
        ),
        compiler_params=pltpu.CompilerParams(
            dimension_semantics=("arbitrary", "arbitrary")),
    )(pred_masks, probs_t, oh)

    numpad = jnp.pad(num[:, :, 0], ((0, 0), (0, QP - Q)))
    denpad = jnp.pad(den[:, :, 0], ((0, 0), (0, QP - Q)),
                     constant_values=1.0)
    fin = _sc_final(vals[:, 0, :], qidx[:, 0, :], numpad, denpad)

    final_scores = fin[:, :K]
    labels_out = labels[:, 0, :K]
    return sem, inst, final_scores, labels_out
